# baseline (speedup 1.0000x reference)
import functools

import jax
import jax.numpy as jnp
from jax import lax
from jax.experimental import pallas as pl

E = 8
D = 9216
H = 128
O = 10
B = 512
BB = 16


def _conv_gate_kernel(x_ref, w1r_ref, b1_ref, w2r_ref, b2_ref, wg_ref,
                      t_ref, logits_ref):
    x = x_ref[...]
    p1 = jnp.concatenate(
        [x[:, dy:dy + 26, dx:dx + 26][..., None]
         for dy in range(3) for dx in range(3)], axis=-1)
    h1 = jnp.dot(p1.reshape(BB * 676, 9), w1r_ref[...],
                 preferred_element_type=jnp.float32)
    h1 = jnp.maximum(h1 + b1_ref[...], 0.0).reshape(BB, 26, 26, 32)
    p2 = jnp.concatenate(
        [h1[:, dy:dy + 24, dx:dx + 24, :]
         for dy in range(3) for dx in range(3)], axis=-1)
    h2 = jnp.dot(p2.reshape(BB * 576, 288), w2r_ref[...],
                 preferred_element_type=jnp.float32)
    h2 = jnp.maximum(h2 + b2_ref[...], 0.0).reshape(BB, 24, 24, 64)
    h2 = h2.reshape(BB, 12, 2, 24, 64).max(axis=2)
    h2 = h2.reshape(BB, 12, 12, 2, 64).max(axis=3)
    t = h2.transpose(0, 3, 1, 2).reshape(BB, D)
    t_ref[...] = t
    logits_ref[...] = jnp.dot(t, wg_ref[...],
                              preferred_element_type=jnp.float32)


def _expert_kernel(logits_ref, t_ref, w1_ref, b1_ref, w2_ref, b2_ref,
                   out_ref):
    e = pl.program_id(0)
    t = t_ref[...]
    h = jnp.dot(t, w1_ref[0], preferred_element_type=jnp.float32)
    h = jnp.maximum(h + b1_ref[0], 0.0)
    o = jnp.dot(h, w2_ref[0], preferred_element_type=jnp.float32)
    o = o + b2_ref[0]

    logits = logits_ref[...]
    m = jnp.max(logits, axis=1, keepdims=True)
    p = jnp.exp(logits - m)
    p = p / jnp.sum(p, axis=1, keepdims=True)
    iota = lax.broadcasted_iota(jnp.int32, (B, E), 1)
    m1 = jnp.max(p, axis=1, keepdims=True)
    i1 = jnp.min(jnp.where(p == m1, iota, E), axis=1, keepdims=True)
    pm = jnp.where(iota == i1, -1.0, p)
    m2 = jnp.max(pm, axis=1, keepdims=True)
    i2 = jnp.min(jnp.where(pm == m2, iota, E), axis=1, keepdims=True)
    sel = (iota == i1) | (iota == i2)
    wcol = jnp.sum(jnp.where(sel & (iota == e), p, 0.0), axis=1,
                   keepdims=True)
    contrib = wcol * o

    @pl.when(e == 0)
    def _():
        out_ref[...] = contrib

    @pl.when(e > 0)
    def _():
        out_ref[...] += contrib

    @pl.when(e == E - 1)
    def _():
        y = out_ref[...]
        ym = jnp.max(y, axis=1, keepdims=True)
        lse = jnp.log(jnp.sum(jnp.exp(y - ym), axis=1, keepdims=True))
        out_ref[...] = y - ym - lse


@functools.partial(jax.jit, static_argnames=("interpret",))
def kernel(x, conv1_w, conv1_b, conv2_w, conv2_b, wg, w1, b1, w2, b2,
           interpret=False):
    x2 = x.reshape(B, 28, 28)
    w1r = conv1_w.reshape(32, 9).T
    b1r = conv1_b.reshape(1, 32)
    w2r = conv2_w.transpose(2, 3, 1, 0).reshape(288, 64)
    b2r = conv2_b.reshape(1, 64)

    nblk = B // BB
    t, logits = pl.pallas_call(
        _conv_gate_kernel,
        grid=(nblk,),
        in_specs=[
            pl.BlockSpec((BB, 28, 28), lambda i: (i, 0, 0)),
            pl.BlockSpec((9, 32), lambda i: (0, 0)),
            pl.BlockSpec((1, 32), lambda i: (0, 0)),
            pl.BlockSpec((288, 64), lambda i: (0, 0)),
            pl.BlockSpec((1, 64), lambda i: (0, 0)),
            pl.BlockSpec((D, E), lambda i: (0, 0)),
        ],
        out_specs=[
            pl.BlockSpec((BB, D), lambda i: (i, 0)),
            pl.BlockSpec((BB, E), lambda i: (i, 0)),
        ],
        out_shape=[
            jax.ShapeDtypeStruct((B, D), jnp.float32),
            jax.ShapeDtypeStruct((B, E), jnp.float32),
        ],
        interpret=interpret,
    )(x2, w1r, b1r, w2r, b2r, wg)

    out = pl.pallas_call(
        _expert_kernel,
        grid=(E,),
        in_specs=[
            pl.BlockSpec((B, E), lambda e: (0, 0)),
            pl.BlockSpec((B, D), lambda e: (0, 0)),
            pl.BlockSpec((1, D, H), lambda e: (e, 0, 0)),
            pl.BlockSpec((1, 1, H), lambda e: (e, 0, 0)),
            pl.BlockSpec((1, H, O), lambda e: (e, 0, 0)),
            pl.BlockSpec((1, 1, O), lambda e: (e, 0, 0)),
        ],
        out_specs=pl.BlockSpec((B, O), lambda e: (0, 0)),
        out_shape=jax.ShapeDtypeStruct((B, O), jnp.float32),
        interpret=interpret,
    )(logits, t, w1, b1.reshape(E, 1, H), w2, b2.reshape(E, 1, O))
    return out
